# baseline (speedup 1.0000x reference)
import functools

import jax
import jax.numpy as jnp
from jax import lax
from jax.experimental import pallas as pl
from jax.experimental.pallas import tpu as pltpu

_BM = 256


def _chamfer_body(src_ref, tar_ref, acc_ref, comp_ref, cham_ref, bds_ref, *, m, n):
    i = pl.program_id(1)
    nb = pl.num_programs(1)
    s = src_ref[0]
    t = tar_ref[0]

    sb = s.astype(jnp.bfloat16)
    tb = t.astype(jnp.bfloat16)
    qr = jnp.dot(sb, tb, preferred_element_type=jnp.float32)
    qre = jnp.dot(s, t, preferred_element_type=jnp.float32,
                  precision=lax.Precision.HIGHEST)
    q2 = jnp.sum(s * s, axis=1, keepdims=True)
    r2 = jnp.sum(t * t, axis=0, keepdims=True)
    ds = (q2 + r2) - 2.0 * qr
    corr = qre - qr

    bm = ds.shape[0]

    rmin = jnp.min(ds, axis=1, keepdims=True)
    jiota = lax.broadcasted_iota(jnp.int32, (bm, n), 1)
    jidx = jnp.where(ds == rmin, jiota, n)
    jstar = jnp.min(jidx, axis=1, keepdims=True)
    csel = jnp.sum(jnp.where(jidx == jstar, corr, 0.0), axis=1, keepdims=True)
    accv = jnp.sqrt(jnp.maximum(rmin - 2.0 * csel, 0.0))
    acc_ref[0, 0] = accv

    cmin = jnp.min(ds, axis=0, keepdims=True)
    riota = lax.broadcasted_iota(jnp.int32, (bm, n), 0)
    iidx = jnp.where(ds == cmin, riota, bm)
    istar = jnp.min(iidx, axis=0, keepdims=True)
    ccor = jnp.sum(jnp.where(iidx == istar, corr, 0.0), axis=0, keepdims=True)
    cexa = jnp.maximum(cmin - 2.0 * ccor, 0.0)

    @pl.when(i == 0)
    def _init():
        bds_ref[...] = cmin
        comp_ref[0] = cexa
        cham_ref[0, 0, :] = jnp.sum(accv).reshape(1)

    @pl.when(i > 0)
    def _accum():
        upd = cmin < bds_ref[...]
        bds_ref[...] = jnp.where(upd, cmin, bds_ref[...])
        comp_ref[0] = jnp.where(upd, cexa, comp_ref[0])
        cham_ref[0, 0, :] = cham_ref[0, 0, :] + jnp.sum(accv)

    @pl.when(i == nb - 1)
    def _finish():
        comp = jnp.sqrt(comp_ref[0, 0, :])
        comp_ref[0, 0, :] = comp
        cham_ref[0, 0, :] = 0.5 * (cham_ref[0, 0, :] / m + jnp.sum(comp) / n)


def kernel(tar, src):
    b, n, _ = tar.shape
    m = src.shape[1]
    tar_t = jnp.transpose(tar, (0, 2, 1))

    acc, comp, cham = pl.pallas_call(
        functools.partial(_chamfer_body, m=m, n=n),
        grid=(b, m // _BM),
        in_specs=[
            pl.BlockSpec((1, _BM, 3), lambda b_, i: (b_, i, 0)),
            pl.BlockSpec((1, 3, n), lambda b_, i: (b_, 0, 0)),
        ],
        out_specs=[
            pl.BlockSpec((1, 1, _BM, 1), lambda b_, i: (b_, i, 0, 0)),
            pl.BlockSpec((1, 1, n), lambda b_, i: (b_, 0, 0)),
            pl.BlockSpec((1, 1, 1), lambda b_, i: (b_, 0, 0)),
        ],
        out_shape=[
            jax.ShapeDtypeStruct((b, m // _BM, _BM, 1), jnp.float32),
            jax.ShapeDtypeStruct((b, 1, n), jnp.float32),
            jax.ShapeDtypeStruct((b, 1, 1), jnp.float32),
        ],
        scratch_shapes=[pltpu.VMEM((1, n), jnp.float32)],
        compiler_params=pltpu.CompilerParams(
            dimension_semantics=("parallel", "arbitrary"),
        ),
    )(src, tar_t)
    return (acc.reshape(b, m), comp[:, 0, :], cham[:, 0, 0])


# speedup vs baseline: 1.4342x; 1.4342x over previous
import functools

import jax
import jax.numpy as jnp
from jax import lax
from jax.experimental import pallas as pl
from jax.experimental.pallas import tpu as pltpu

_BM = 256


def _chamfer_body(src_ref, tar_ref, acc_ref, comp_ref, cham_ref, bds_ref, *, m, n):
    i = pl.program_id(1)
    nb = pl.num_programs(1)
    s = src_ref[0]
    t = tar_ref[0]

    sb = s.astype(jnp.bfloat16)
    tb = t.astype(jnp.bfloat16)
    qr = jnp.dot(sb, tb, preferred_element_type=jnp.float32)
    slo = (s - sb.astype(jnp.float32)).astype(jnp.bfloat16)
    tlo = (t - tb.astype(jnp.float32)).astype(jnp.bfloat16)
    corr = jnp.dot(
        jnp.concatenate([sb, slo, slo], axis=1),
        jnp.concatenate([tlo, tb, tlo], axis=0),
        preferred_element_type=jnp.float32,
    )
    q2 = jnp.sum(s * s, axis=1, keepdims=True)
    r2 = jnp.sum(t * t, axis=0, keepdims=True)
    ds = (q2 + r2) - 2.0 * qr

    bm = ds.shape[0]

    rmin = jnp.min(ds, axis=1, keepdims=True)
    jiota = lax.broadcasted_iota(jnp.int32, (bm, n), 1)
    jidx = jnp.where(ds == rmin, jiota, n)
    jstar = jnp.min(jidx, axis=1, keepdims=True)
    csel = jnp.sum(jnp.where(jidx == jstar, corr, 0.0), axis=1, keepdims=True)
    accv = jnp.sqrt(jnp.maximum(rmin - 2.0 * csel, 0.0))
    acc_ref[0, 0] = accv

    cmin = jnp.min(ds, axis=0, keepdims=True)
    riota = lax.broadcasted_iota(jnp.int32, (bm, n), 0)
    iidx = jnp.where(ds == cmin, riota, bm)
    istar = jnp.min(iidx, axis=0, keepdims=True)
    ccor = jnp.sum(jnp.where(iidx == istar, corr, 0.0), axis=0, keepdims=True)
    cexa = jnp.maximum(cmin - 2.0 * ccor, 0.0)

    @pl.when(i == 0)
    def _init():
        bds_ref[...] = cmin
        comp_ref[0] = cexa
        cham_ref[0, 0, :] = jnp.sum(accv).reshape(1)

    @pl.when(i > 0)
    def _accum():
        upd = cmin < bds_ref[...]
        bds_ref[...] = jnp.where(upd, cmin, bds_ref[...])
        comp_ref[0] = jnp.where(upd, cexa, comp_ref[0])
        cham_ref[0, 0, :] = cham_ref[0, 0, :] + jnp.sum(accv)

    @pl.when(i == nb - 1)
    def _finish():
        comp = jnp.sqrt(comp_ref[0, 0, :])
        comp_ref[0, 0, :] = comp
        cham_ref[0, 0, :] = 0.5 * (cham_ref[0, 0, :] / m + jnp.sum(comp) / n)


def kernel(tar, src):
    b, n, _ = tar.shape
    m = src.shape[1]
    tar_t = jnp.transpose(tar, (0, 2, 1))

    acc, comp, cham = pl.pallas_call(
        functools.partial(_chamfer_body, m=m, n=n),
        grid=(b, m // _BM),
        in_specs=[
            pl.BlockSpec((1, _BM, 3), lambda b_, i: (b_, i, 0)),
            pl.BlockSpec((1, 3, n), lambda b_, i: (b_, 0, 0)),
        ],
        out_specs=[
            pl.BlockSpec((1, 1, _BM, 1), lambda b_, i: (b_, i, 0, 0)),
            pl.BlockSpec((1, 1, n), lambda b_, i: (b_, 0, 0)),
            pl.BlockSpec((1, 1, 1), lambda b_, i: (b_, 0, 0)),
        ],
        out_shape=[
            jax.ShapeDtypeStruct((b, m // _BM, _BM, 1), jnp.float32),
            jax.ShapeDtypeStruct((b, 1, n), jnp.float32),
            jax.ShapeDtypeStruct((b, 1, 1), jnp.float32),
        ],
        scratch_shapes=[pltpu.VMEM((1, n), jnp.float32)],
        compiler_params=pltpu.CompilerParams(
            dimension_semantics=("parallel", "arbitrary"),
        ),
    )(src, tar_t)
    return (acc.reshape(b, m), comp[:, 0, :], cham[:, 0, 0])


# speedup vs baseline: 1.5557x; 1.0847x over previous
import functools

import jax
import jax.numpy as jnp
from jax import lax
from jax.experimental import pallas as pl
from jax.experimental.pallas import tpu as pltpu

_BM = 256


def _chamfer_body(src_ref, tar_ref, acc_ref, comp_ref, cham_ref, bds_ref, *, m, n):
    i = pl.program_id(1)
    nb = pl.num_programs(1)
    s = src_ref[0]
    t = tar_ref[0]

    sb = s.astype(jnp.bfloat16)
    tb = t.astype(jnp.bfloat16)
    qr = jnp.dot(sb, tb, preferred_element_type=jnp.float32)
    slo = (s - sb.astype(jnp.float32)).astype(jnp.bfloat16)
    tlo = (t - tb.astype(jnp.float32)).astype(jnp.bfloat16)
    corr = jnp.dot(
        jnp.concatenate([sb, slo, slo], axis=1),
        jnp.concatenate([tlo, tb, tlo], axis=0),
        preferred_element_type=jnp.float32,
    )
    q2 = jnp.sum(s * s, axis=1, keepdims=True)
    r2 = jnp.sum(t * t, axis=0, keepdims=True)
    ds = (q2 + r2) - 2.0 * qr

    bm = ds.shape[0]

    rmin = jnp.min(ds, axis=1, keepdims=True)
    jiota = lax.broadcasted_iota(jnp.int32, (bm, n), 1)
    jstar = jnp.argmin(ds, axis=1).astype(jnp.int32)[:, None]
    csel = jnp.sum(jnp.where(jiota == jstar, corr, 0.0), axis=1, keepdims=True)
    accv = jnp.sqrt(jnp.maximum(rmin - 2.0 * csel, 0.0))
    acc_ref[0, 0] = accv

    cmin = jnp.min(ds, axis=0, keepdims=True)
    riota = lax.broadcasted_iota(jnp.int32, (bm, n), 0)
    istar = jnp.argmin(ds, axis=0).astype(jnp.int32)[None, :]
    ccor = jnp.sum(jnp.where(riota == istar, corr, 0.0), axis=0, keepdims=True)
    cexa = jnp.maximum(cmin - 2.0 * ccor, 0.0)

    @pl.when(i == 0)
    def _init():
        bds_ref[...] = cmin
        comp_ref[0] = cexa
        cham_ref[0, 0, :] = jnp.sum(accv).reshape(1)

    @pl.when(i > 0)
    def _accum():
        upd = cmin < bds_ref[...]
        bds_ref[...] = jnp.where(upd, cmin, bds_ref[...])
        comp_ref[0] = jnp.where(upd, cexa, comp_ref[0])
        cham_ref[0, 0, :] = cham_ref[0, 0, :] + jnp.sum(accv)

    @pl.when(i == nb - 1)
    def _finish():
        comp = jnp.sqrt(comp_ref[0, 0, :])
        comp_ref[0, 0, :] = comp
        cham_ref[0, 0, :] = 0.5 * (cham_ref[0, 0, :] / m + jnp.sum(comp) / n)


def kernel(tar, src):
    b, n, _ = tar.shape
    m = src.shape[1]
    tar_t = jnp.transpose(tar, (0, 2, 1))

    acc, comp, cham = pl.pallas_call(
        functools.partial(_chamfer_body, m=m, n=n),
        grid=(b, m // _BM),
        in_specs=[
            pl.BlockSpec((1, _BM, 3), lambda b_, i: (b_, i, 0)),
            pl.BlockSpec((1, 3, n), lambda b_, i: (b_, 0, 0)),
        ],
        out_specs=[
            pl.BlockSpec((1, 1, _BM, 1), lambda b_, i: (b_, i, 0, 0)),
            pl.BlockSpec((1, 1, n), lambda b_, i: (b_, 0, 0)),
            pl.BlockSpec((1, 1, 1), lambda b_, i: (b_, 0, 0)),
        ],
        out_shape=[
            jax.ShapeDtypeStruct((b, m // _BM, _BM, 1), jnp.float32),
            jax.ShapeDtypeStruct((b, 1, n), jnp.float32),
            jax.ShapeDtypeStruct((b, 1, 1), jnp.float32),
        ],
        scratch_shapes=[pltpu.VMEM((1, n), jnp.float32)],
        compiler_params=pltpu.CompilerParams(
            dimension_semantics=("parallel", "arbitrary"),
        ),
    )(src, tar_t)
    return (acc.reshape(b, m), comp[:, 0, :], cham[:, 0, 0])


# speedup vs baseline: 1.8902x; 1.2150x over previous
import functools

import jax
import jax.numpy as jnp
from jax import lax
from jax.experimental import pallas as pl
from jax.experimental.pallas import tpu as pltpu

_BM = 1024


def _chamfer_body(src_ref, tar_ref, acc_ref, comp_ref, cham_ref, bds_ref, *, m, n):
    i = pl.program_id(1)
    nb = pl.num_programs(1)
    s = src_ref[0]
    t = tar_ref[0]

    sb = s.astype(jnp.bfloat16)
    tb = t.astype(jnp.bfloat16)
    qr = jnp.dot(sb, tb, preferred_element_type=jnp.float32)
    slo = (s - sb.astype(jnp.float32)).astype(jnp.bfloat16)
    tlo = (t - tb.astype(jnp.float32)).astype(jnp.bfloat16)
    corr = jnp.dot(
        jnp.concatenate([sb, slo, slo], axis=1),
        jnp.concatenate([tlo, tb, tlo], axis=0),
        preferred_element_type=jnp.float32,
    )
    q2 = jnp.sum(s * s, axis=1, keepdims=True)
    r2 = jnp.sum(t * t, axis=0, keepdims=True)
    ds = (q2 + r2) - 2.0 * qr

    bm = ds.shape[0]

    rmin = jnp.min(ds, axis=1, keepdims=True)
    jiota = lax.broadcasted_iota(jnp.int32, (bm, n), 1)
    jstar = jnp.argmin(ds, axis=1).astype(jnp.int32)[:, None]
    csel = jnp.sum(jnp.where(jiota == jstar, corr, 0.0), axis=1, keepdims=True)
    accv = jnp.sqrt(jnp.maximum(rmin - 2.0 * csel, 0.0))
    acc_ref[0, 0] = accv

    cmin = jnp.min(ds, axis=0, keepdims=True)
    riota = lax.broadcasted_iota(jnp.int32, (bm, n), 0)
    istar = jnp.argmin(ds, axis=0).astype(jnp.int32)[None, :]
    ccor = jnp.sum(jnp.where(riota == istar, corr, 0.0), axis=0, keepdims=True)
    cexa = jnp.maximum(cmin - 2.0 * ccor, 0.0)

    @pl.when(i == 0)
    def _init():
        bds_ref[...] = cmin
        comp_ref[0] = cexa
        cham_ref[0, 0, :] = jnp.sum(accv).reshape(1)

    @pl.when(i > 0)
    def _accum():
        upd = cmin < bds_ref[...]
        bds_ref[...] = jnp.where(upd, cmin, bds_ref[...])
        comp_ref[0] = jnp.where(upd, cexa, comp_ref[0])
        cham_ref[0, 0, :] = cham_ref[0, 0, :] + jnp.sum(accv)

    @pl.when(i == nb - 1)
    def _finish():
        comp = jnp.sqrt(comp_ref[0, 0, :])
        comp_ref[0, 0, :] = comp
        cham_ref[0, 0, :] = 0.5 * (cham_ref[0, 0, :] / m + jnp.sum(comp) / n)


def kernel(tar, src):
    b, n, _ = tar.shape
    m = src.shape[1]
    tar_t = jnp.transpose(tar, (0, 2, 1))

    acc, comp, cham = pl.pallas_call(
        functools.partial(_chamfer_body, m=m, n=n),
        grid=(b, m // _BM),
        in_specs=[
            pl.BlockSpec((1, _BM, 3), lambda b_, i: (b_, i, 0)),
            pl.BlockSpec((1, 3, n), lambda b_, i: (b_, 0, 0)),
        ],
        out_specs=[
            pl.BlockSpec((1, 1, _BM, 1), lambda b_, i: (b_, i, 0, 0)),
            pl.BlockSpec((1, 1, n), lambda b_, i: (b_, 0, 0)),
            pl.BlockSpec((1, 1, 1), lambda b_, i: (b_, 0, 0)),
        ],
        out_shape=[
            jax.ShapeDtypeStruct((b, m // _BM, _BM, 1), jnp.float32),
            jax.ShapeDtypeStruct((b, 1, n), jnp.float32),
            jax.ShapeDtypeStruct((b, 1, 1), jnp.float32),
        ],
        scratch_shapes=[pltpu.VMEM((1, n), jnp.float32)],
        compiler_params=pltpu.CompilerParams(
            dimension_semantics=("parallel", "arbitrary"),
        ),
    )(src, tar_t)
    return (acc.reshape(b, m), comp[:, 0, :], cham[:, 0, 0])


# speedup vs baseline: 1.9286x; 1.0203x over previous
import functools

import jax
import jax.numpy as jnp
from jax import lax
from jax.experimental import pallas as pl
from jax.experimental.pallas import tpu as pltpu

_BM = 1024


def _chamfer_body(src_ref, tar_ref, acc_ref, comp_ref, cham_ref, bds_ref, *, m, n):
    i = pl.program_id(1)
    nb = pl.num_programs(1)
    s = src_ref[0]
    t = tar_ref[0]

    sb = s.astype(jnp.bfloat16)
    tb = t.astype(jnp.bfloat16)
    qr = jnp.dot(sb, tb, preferred_element_type=jnp.float32)
    slo = (s - sb.astype(jnp.float32)).astype(jnp.bfloat16)
    tlo = (t - tb.astype(jnp.float32)).astype(jnp.bfloat16)
    corr = jnp.dot(
        jnp.concatenate([sb, slo], axis=1),
        jnp.concatenate([tlo, tb], axis=0),
        preferred_element_type=jnp.float32,
    )
    q2 = jnp.sum(s * s, axis=1, keepdims=True)
    r2 = jnp.sum(t * t, axis=0, keepdims=True)
    ds = (q2 + r2) - 2.0 * qr

    bm = ds.shape[0]

    rmin = jnp.min(ds, axis=1, keepdims=True)
    jiota = lax.broadcasted_iota(jnp.int32, (bm, n), 1)
    jstar = jnp.argmin(ds, axis=1).astype(jnp.int32)[:, None]
    csel = jnp.sum(jnp.where(jiota == jstar, corr, 0.0), axis=1, keepdims=True)
    accv = jnp.sqrt(jnp.maximum(rmin - 2.0 * csel, 0.0))
    acc_ref[0, 0] = accv

    cmin = jnp.min(ds, axis=0, keepdims=True)
    riota = lax.broadcasted_iota(jnp.int32, (bm, n), 0)
    istar = jnp.argmin(ds, axis=0).astype(jnp.int32)[None, :]
    ccor = jnp.sum(jnp.where(riota == istar, corr, 0.0), axis=0, keepdims=True)
    cexa = jnp.maximum(cmin - 2.0 * ccor, 0.0)

    @pl.when(i == 0)
    def _init():
        bds_ref[...] = cmin
        comp_ref[0] = cexa
        cham_ref[0, 0, :] = jnp.sum(accv).reshape(1)

    @pl.when(i > 0)
    def _accum():
        upd = cmin < bds_ref[...]
        bds_ref[...] = jnp.where(upd, cmin, bds_ref[...])
        comp_ref[0] = jnp.where(upd, cexa, comp_ref[0])
        cham_ref[0, 0, :] = cham_ref[0, 0, :] + jnp.sum(accv)

    @pl.when(i == nb - 1)
    def _finish():
        comp = jnp.sqrt(comp_ref[0, 0, :])
        comp_ref[0, 0, :] = comp
        cham_ref[0, 0, :] = 0.5 * (cham_ref[0, 0, :] / m + jnp.sum(comp) / n)


def kernel(tar, src):
    b, n, _ = tar.shape
    m = src.shape[1]
    tar_t = jnp.transpose(tar, (0, 2, 1))

    acc, comp, cham = pl.pallas_call(
        functools.partial(_chamfer_body, m=m, n=n),
        grid=(b, m // _BM),
        in_specs=[
            pl.BlockSpec((1, _BM, 3), lambda b_, i: (b_, i, 0)),
            pl.BlockSpec((1, 3, n), lambda b_, i: (b_, 0, 0)),
        ],
        out_specs=[
            pl.BlockSpec((1, 1, _BM, 1), lambda b_, i: (b_, i, 0, 0)),
            pl.BlockSpec((1, 1, n), lambda b_, i: (b_, 0, 0)),
            pl.BlockSpec((1, 1, 1), lambda b_, i: (b_, 0, 0)),
        ],
        out_shape=[
            jax.ShapeDtypeStruct((b, m // _BM, _BM, 1), jnp.float32),
            jax.ShapeDtypeStruct((b, 1, n), jnp.float32),
            jax.ShapeDtypeStruct((b, 1, 1), jnp.float32),
        ],
        scratch_shapes=[pltpu.VMEM((1, n), jnp.float32)],
        compiler_params=pltpu.CompilerParams(
            dimension_semantics=("parallel", "arbitrary"),
        ),
    )(src, tar_t)
    return (acc.reshape(b, m), comp[:, 0, :], cham[:, 0, 0])


# speedup vs baseline: 1.9988x; 1.0364x over previous
import functools

import jax
import jax.numpy as jnp
from jax import lax
from jax.experimental import pallas as pl
from jax.experimental.pallas import tpu as pltpu

_BM = 2048


def _chamfer_body(src_ref, tar_ref, acc_ref, comp_ref, cham_ref, bds_ref, *, m, n):
    i = pl.program_id(1)
    nb = pl.num_programs(1)
    s = src_ref[0]
    t = tar_ref[0]

    sb = s.astype(jnp.bfloat16)
    tb = t.astype(jnp.bfloat16)
    qr = jnp.dot(sb, tb, preferred_element_type=jnp.float32)
    slo = (s - sb.astype(jnp.float32)).astype(jnp.bfloat16)
    tlo = (t - tb.astype(jnp.float32)).astype(jnp.bfloat16)
    corr = jnp.dot(
        jnp.concatenate([sb, slo], axis=1),
        jnp.concatenate([tlo, tb], axis=0),
        preferred_element_type=jnp.float32,
    )
    q2 = jnp.sum(s * s, axis=1, keepdims=True)
    r2 = jnp.sum(t * t, axis=0, keepdims=True)
    ds = (q2 + r2) - 2.0 * qr

    bm = ds.shape[0]

    rmin = jnp.min(ds, axis=1, keepdims=True)
    jiota = lax.broadcasted_iota(jnp.int32, (bm, n), 1)
    jstar = jnp.argmin(ds, axis=1).astype(jnp.int32)[:, None]
    csel = jnp.sum(jnp.where(jiota == jstar, corr, 0.0), axis=1, keepdims=True)
    accv = jnp.sqrt(jnp.maximum(rmin - 2.0 * csel, 0.0))
    acc_ref[0, 0] = accv

    cmin = jnp.min(ds, axis=0, keepdims=True)
    riota = lax.broadcasted_iota(jnp.int32, (bm, n), 0)
    istar = jnp.argmin(ds, axis=0).astype(jnp.int32)[None, :]
    ccor = jnp.sum(jnp.where(riota == istar, corr, 0.0), axis=0, keepdims=True)
    cexa = jnp.maximum(cmin - 2.0 * ccor, 0.0)

    @pl.when(i == 0)
    def _init():
        bds_ref[...] = cmin
        comp_ref[0] = cexa
        cham_ref[0, 0, :] = jnp.sum(accv).reshape(1)

    @pl.when(i > 0)
    def _accum():
        upd = cmin < bds_ref[...]
        bds_ref[...] = jnp.where(upd, cmin, bds_ref[...])
        comp_ref[0] = jnp.where(upd, cexa, comp_ref[0])
        cham_ref[0, 0, :] = cham_ref[0, 0, :] + jnp.sum(accv)

    @pl.when(i == nb - 1)
    def _finish():
        comp = jnp.sqrt(comp_ref[0, 0, :])
        comp_ref[0, 0, :] = comp
        cham_ref[0, 0, :] = 0.5 * (cham_ref[0, 0, :] / m + jnp.sum(comp) / n)


def kernel(tar, src):
    b, n, _ = tar.shape
    m = src.shape[1]
    tar_t = jnp.transpose(tar, (0, 2, 1))

    acc, comp, cham = pl.pallas_call(
        functools.partial(_chamfer_body, m=m, n=n),
        grid=(b, m // _BM),
        in_specs=[
            pl.BlockSpec((1, _BM, 3), lambda b_, i: (b_, i, 0)),
            pl.BlockSpec((1, 3, n), lambda b_, i: (b_, 0, 0)),
        ],
        out_specs=[
            pl.BlockSpec((1, 1, _BM, 1), lambda b_, i: (b_, i, 0, 0)),
            pl.BlockSpec((1, 1, n), lambda b_, i: (b_, 0, 0)),
            pl.BlockSpec((1, 1, 1), lambda b_, i: (b_, 0, 0)),
        ],
        out_shape=[
            jax.ShapeDtypeStruct((b, m // _BM, _BM, 1), jnp.float32),
            jax.ShapeDtypeStruct((b, 1, n), jnp.float32),
            jax.ShapeDtypeStruct((b, 1, 1), jnp.float32),
        ],
        scratch_shapes=[pltpu.VMEM((1, n), jnp.float32)],
        compiler_params=pltpu.CompilerParams(
            dimension_semantics=("parallel", "arbitrary"),
        ),
    )(src, tar_t)
    return (acc.reshape(b, m), comp[:, 0, :], cham[:, 0, 0])
